# baseline (speedup 1.0000x reference)
import functools

import jax
import jax.numpy as jnp
from jax import lax
from jax.experimental import pallas as pl
from jax.experimental.pallas import tpu as pltpu
from jax.experimental.pallas import tpu_sc as plsc

N = 10000
E = 320000
D = 128
DE = 16
H = 2 * D
L = 2

NC = 2
NS = 16
NW = NC * NS
EW = E // NW
K = 40
NCHUNK = EW // K
SLOTS = 4
ISLOTS = 8
PEEL_LO, PEEL_HI = -4, 8
TAIL_LO, TAIL_HI = 240, 252
NGROUPS = (TAIL_LO - PEEL_HI) // ISLOTS
ZROWS = 640
AGG_ROWS = NS * ZROWS


def _sc_message_aggregate(h, e, src, dst, zeros):
  mesh = plsc.VectorSubcoreMesh(core_axis_name="c", subcore_axis_name="s")

  @functools.partial(
      pl.kernel,
      mesh=mesh,
      out_type=jax.ShapeDtypeStruct((2 * N, D), jnp.float32),
      scratch_types=[
          pltpu.VMEM((ISLOTS, 2, K), jnp.int32),
          pltpu.VMEM((SLOTS, K, D), jnp.float32),
          pltpu.VMEM((SLOTS, K, D), jnp.float32),
          pltpu.VMEM_SHARED((AGG_ROWS, D), jnp.float32),
          pltpu.SemaphoreType.DMA((ISLOTS,)),
          pltpu.SemaphoreType.DMA((SLOTS,)),
          pltpu.SemaphoreType.DMA((SLOTS,)),
          pltpu.SemaphoreType.DMA((SLOTS,)),
      ],
  )
  def k(h_hbm, e_hbm, src_hbm, dst_hbm, zero_hbm, out_hbm,
        idx_v, e_v, h_v, agg, sem_i, sem_e, sem_g, sem_s):
    c = lax.axis_index("c")
    s = lax.axis_index("s")
    wid = c * NS + s

    pltpu.sync_copy(zero_hbm, agg.at[pl.ds(s * ZROWS, ZROWS)])
    plsc.subcore_barrier()

    base = wid * EW

    def idx_start(j, m):
      off = base + j * K
      pltpu.async_copy(src_hbm.at[pl.ds(off, K)], idx_v.at[m % ISLOTS, 0],
                       sem_i.at[m % ISLOTS])
      pltpu.async_copy(dst_hbm.at[pl.ds(off, K)], idx_v.at[m % ISLOTS, 1],
                       sem_i.at[m % ISLOTS])

    def idx_wait(m):
      pltpu.make_async_copy(src_hbm.at[pl.ds(0, K)],
                            idx_v.at[m % ISLOTS, 0],
                            sem_i.at[m % ISLOTS]).wait()
      pltpu.make_async_copy(dst_hbm.at[pl.ds(0, K)],
                            idx_v.at[m % ISLOTS, 1],
                            sem_i.at[m % ISLOTS]).wait()

    def e_start(j, m):
      pltpu.async_copy(e_hbm.at[pl.ds(base + j * K, K)], e_v.at[m % SLOTS],
                       sem_e.at[m % SLOTS])

    def e_wait(m):
      pltpu.make_async_copy(e_hbm.at[pl.ds(0, K)], e_v.at[m % SLOTS],
                            sem_e.at[m % SLOTS]).wait()

    def gather_start(m):
      pltpu.async_copy(h_hbm.at[idx_v.at[m % ISLOTS, 0]], h_v.at[m % SLOTS],
                       sem_g.at[m % SLOTS])

    def gather_wait(m):
      pltpu.make_async_copy(h_hbm.at[idx_v.at[m % ISLOTS, 0]],
                            h_v.at[m % SLOTS], sem_g.at[m % SLOTS]).wait()

    def scatter_start(m):
      pltpu.async_copy(h_v.at[m % SLOTS], agg.at[idx_v.at[m % ISLOTS, 1]],
                       sem_s.at[m % SLOTS], add=True)

    def scatter_wait(m):
      pltpu.make_async_copy(h_v.at[m % SLOTS],
                            agg.at[idx_v.at[m % ISLOTS, 1]],
                            sem_s.at[m % SLOTS]).wait()

    def compute(m, unroll=4):
      b = m % SLOTS

      @plsc.parallel_loop(0, K, 1, unroll=unroll)
      def _(i):
        for r in range(D // 16):
          sl = pl.ds(r * 16, 16)
          h_v[b, i, sl] = jnp.maximum(h_v[b, i, sl] + e_v[b, i, sl], 0.0)

    def step(j, m, guard):
      def ok(chunk):
        return guard is None or 0 <= chunk < NCHUNK

      if ok(j - 2):
        scatter_wait(m - 2)
      if ok(j + 4):
        idx_start(j + 4, m + 4)
      if ok(j + 2):
        e_start(j + 2, m + 2)
        idx_wait(m + 2)
        gather_start(m + 2)
      if ok(j):
        gather_wait(m)
        e_wait(m)
        compute(m, unroll=8 if guard is None else 1)
        scatter_start(m)

    for t in range(PEEL_LO, PEEL_HI):
      step(t, t % ISLOTS, guard=t)

    def group(g, carry):
      j0 = PEEL_HI + ISLOTS * g
      for i in range(ISLOTS):
        step(j0 + i, (PEEL_HI + i) % ISLOTS, guard=None)
      return carry

    lax.fori_loop(0, NGROUPS, group, 0)

    for t in range(TAIL_LO, TAIL_HI):
      step(t, t % ISLOTS, guard=t)

    plsc.subcore_barrier()

    @pl.when(s < NS - 1)
    def _():
      pltpu.sync_copy(agg.at[pl.ds(s * ZROWS, ZROWS)],
                      out_hbm.at[pl.ds(c * N + s * ZROWS, ZROWS)])

    @pl.when(s == NS - 1)
    def _():
      last = N - (NS - 1) * ZROWS
      pltpu.sync_copy(agg.at[pl.ds((NS - 1) * ZROWS, last)],
                      out_hbm.at[pl.ds(c * N + (NS - 1) * ZROWS, last)])

  return k(h, e, src, dst, zeros)


_G = 128 // DE
_BER = 2000


def _embed_body(ea_ref, w_ref, b_ref, out_ref):
  z = (jnp.dot(ea_ref[...], w_ref[...], preferred_element_type=jnp.float32)
       + b_ref[...])
  out_ref[...] = z.reshape(_G * _BER, D)


def _tc_edge_embed(ea8, w8, b8):
  return pl.pallas_call(
      _embed_body,
      grid=(E // _G // _BER,),
      in_specs=[
          pl.BlockSpec((_BER, _G * DE), lambda i: (i, 0)),
          pl.BlockSpec((_G * DE, _G * D), lambda i: (0, 0)),
          pl.BlockSpec((1, _G * D), lambda i: (0, 0)),
      ],
      out_specs=pl.BlockSpec((_G * _BER, D), lambda i: (i, 0)),
      out_shape=jax.ShapeDtypeStruct((E, D), jnp.float32),
  )(ea8, w8, b8.reshape(1, _G * D))


def _mlp_bn_body(relu_out, h_ref, a0_ref, a1_ref, sc_ref, w1_ref, b1_ref,
                 w2_ref, b2_ref, g_ref, bt_ref, out_ref):
  zin = sc_ref[...] * h_ref[...] + a0_ref[...] + a1_ref[...]
  t = jnp.maximum(
      jnp.dot(zin, w1_ref[...], preferred_element_type=jnp.float32)
      + b1_ref[...], 0.0)
  z = (jnp.dot(t, w2_ref[...], preferred_element_type=jnp.float32)
       + b2_ref[...])
  mean = jnp.mean(z, axis=0, keepdims=True)
  var = jnp.mean((z - mean) ** 2, axis=0, keepdims=True)
  zn = (z - mean) * lax.rsqrt(var + 1e-5) * g_ref[...] + bt_ref[...]
  if relu_out:
    zn = jnp.maximum(zn, 0.0)
  out_ref[...] = zn


def _tc_mlp_bn(h, agg2, scale, w1, b1, w2, b2, gamma, beta, relu_out):
  full = lambda shape: pl.BlockSpec(shape, lambda g: tuple(0 for _ in shape))
  return pl.pallas_call(
      functools.partial(_mlp_bn_body, relu_out),
      grid=(1,),
      in_specs=[
          full((N, D)),
          pl.BlockSpec((N, D), lambda g: (0, 0)),
          pl.BlockSpec((N, D), lambda g: (1, 0)),
          full((1, D)), full((D, H)), full((1, H)), full((H, D)),
          full((1, D)), full((1, D)), full((1, D)),
      ],
      out_specs=full((N, D)),
      out_shape=jax.ShapeDtypeStruct((N, D), jnp.float32),
  )(h, agg2, agg2,
    jnp.broadcast_to(scale.reshape(1, 1), (1, D)),
    w1, b1.reshape(1, H), w2, b2.reshape(1, D),
    gamma.reshape(1, D), beta.reshape(1, D))


def kernel(x, edge_index, edge_attr, W_e, b_e, eps, W1, b1, W2, b2,
           gamma, beta):
  zeros = jnp.zeros((ZROWS, D), dtype=jnp.float32)
  src = edge_index[0]
  dst = edge_index[1]

  ea8 = edge_attr.reshape(E // _G, _G * DE)
  eye8 = jnp.eye(_G, dtype=jnp.float32)
  h = x
  for l in range(L):
    w8_l = jnp.einsum("jm,kd->jkmd", eye8, W_e[l]).reshape(_G * DE, _G * D)
    b8_l = jnp.tile(b_e[l], _G)
    e = _tc_edge_embed(ea8, w8_l, b8_l)
    agg2 = _sc_message_aggregate(h, e, src, dst, zeros)
    h = _tc_mlp_bn(h, agg2, 1.0 + eps[l],
                   W1[l], b1[l], W2[l], b2[l], gamma[l], beta[l],
                   relu_out=(l < L - 1))
  return h


# speedup vs baseline: 1.0665x; 1.0665x over previous
import functools

import jax
import jax.numpy as jnp
from jax import lax
from jax.experimental import pallas as pl
from jax.experimental.pallas import tpu as pltpu
from jax.experimental.pallas import tpu_sc as plsc

N = 10000
E = 320000
D = 128
DE = 16
H = 2 * D
L = 2

NC = 2
NS = 16
NW = NC * NS
EW = E // NW
K = 40
NCHUNK = EW // K
SLOTS = 4
ISLOTS = 8
PEEL_LO, PEEL_HI = -4, 8
TAIL_LO, TAIL_HI = 240, 252
NGROUPS = (TAIL_LO - PEEL_HI) // ISLOTS
ZROWS = 640
AGG_ROWS = NS * ZROWS


def _sc_message_aggregate(h, e, src, dst, zeros):
  mesh = plsc.VectorSubcoreMesh(core_axis_name="c", subcore_axis_name="s")

  @functools.partial(
      pl.kernel,
      mesh=mesh,
      out_type=jax.ShapeDtypeStruct((2 * N, D), jnp.float32),
      scratch_types=[
          pltpu.VMEM((ISLOTS, 2, K), jnp.int32),
          pltpu.VMEM((SLOTS, K, D), jnp.float32),
          pltpu.VMEM((SLOTS, K, D), jnp.float32),
          pltpu.VMEM_SHARED((AGG_ROWS, D), jnp.float32),
          pltpu.SemaphoreType.DMA((ISLOTS,)),
          pltpu.SemaphoreType.DMA((SLOTS,)),
          pltpu.SemaphoreType.DMA((SLOTS,)),
          pltpu.SemaphoreType.DMA((SLOTS,)),
      ],
  )
  def k(h_hbm, e_hbm, src_hbm, dst_hbm, zero_hbm, out_hbm,
        idx_v, e_v, h_v, agg, sem_i, sem_e, sem_g, sem_s):
    c = lax.axis_index("c")
    s = lax.axis_index("s")
    wid = c * NS + s

    pltpu.sync_copy(zero_hbm, agg.at[pl.ds(s * ZROWS, ZROWS)])
    plsc.subcore_barrier()

    base = wid * EW

    def idx_start(j, m):
      off = base + j * K
      pltpu.async_copy(src_hbm.at[pl.ds(off, K)], idx_v.at[m % ISLOTS, 0],
                       sem_i.at[m % ISLOTS])
      pltpu.async_copy(dst_hbm.at[pl.ds(off, K)], idx_v.at[m % ISLOTS, 1],
                       sem_i.at[m % ISLOTS])

    def idx_wait(m):
      pltpu.make_async_copy(src_hbm.at[pl.ds(0, K)],
                            idx_v.at[m % ISLOTS, 0],
                            sem_i.at[m % ISLOTS]).wait()
      pltpu.make_async_copy(dst_hbm.at[pl.ds(0, K)],
                            idx_v.at[m % ISLOTS, 1],
                            sem_i.at[m % ISLOTS]).wait()

    def e_start(j, m):
      pltpu.async_copy(e_hbm.at[pl.ds(base + j * K, K)], e_v.at[m % SLOTS],
                       sem_e.at[m % SLOTS])

    def e_wait(m):
      pltpu.make_async_copy(e_hbm.at[pl.ds(0, K)], e_v.at[m % SLOTS],
                            sem_e.at[m % SLOTS]).wait()

    def gather_start(m):
      pltpu.async_copy(h_hbm.at[idx_v.at[m % ISLOTS, 0]], h_v.at[m % SLOTS],
                       sem_g.at[m % SLOTS])

    def gather_wait(m):
      pltpu.make_async_copy(h_hbm.at[idx_v.at[m % ISLOTS, 0]],
                            h_v.at[m % SLOTS], sem_g.at[m % SLOTS]).wait()

    def scatter_start(m):
      pltpu.async_copy(h_v.at[m % SLOTS], agg.at[idx_v.at[m % ISLOTS, 1]],
                       sem_s.at[m % SLOTS], add=True)

    def scatter_wait(m):
      pltpu.make_async_copy(h_v.at[m % SLOTS],
                            agg.at[idx_v.at[m % ISLOTS, 1]],
                            sem_s.at[m % SLOTS]).wait()

    def compute(m, unroll=4):
      b = m % SLOTS

      @plsc.parallel_loop(0, K, 1, unroll=unroll)
      def _(i):
        for r in range(D // 16):
          sl = pl.ds(r * 16, 16)
          h_v[b, i, sl] = jnp.maximum(h_v[b, i, sl] + e_v[b, i, sl], 0.0)

    def step(j, m, guard):
      def ok(chunk):
        return guard is None or 0 <= chunk < NCHUNK

      if ok(j - 2):
        scatter_wait(m - 2)
      if ok(j + 4):
        idx_start(j + 4, m + 4)
      if ok(j + 2):
        e_start(j + 2, m + 2)
        idx_wait(m + 2)
        gather_start(m + 2)
      if ok(j):
        gather_wait(m)
        e_wait(m)
        compute(m, unroll=2 if guard is None else 1)
        scatter_start(m)

    for t in range(PEEL_LO, PEEL_HI):
      step(t, t % ISLOTS, guard=t)

    def group(g, carry):
      j0 = PEEL_HI + ISLOTS * g
      for i in range(ISLOTS):
        step(j0 + i, (PEEL_HI + i) % ISLOTS, guard=None)
      return carry

    lax.fori_loop(0, NGROUPS, group, 0)

    for t in range(TAIL_LO, TAIL_HI):
      step(t, t % ISLOTS, guard=t)

    plsc.subcore_barrier()

    @pl.when(s < NS - 1)
    def _():
      pltpu.sync_copy(agg.at[pl.ds(s * ZROWS, ZROWS)],
                      out_hbm.at[pl.ds(c * N + s * ZROWS, ZROWS)])

    @pl.when(s == NS - 1)
    def _():
      last = N - (NS - 1) * ZROWS
      pltpu.sync_copy(agg.at[pl.ds((NS - 1) * ZROWS, last)],
                      out_hbm.at[pl.ds(c * N + (NS - 1) * ZROWS, last)])

  return k(h, e, src, dst, zeros)


_G = 128 // DE
_BER = 2000


def _embed_body(ea_ref, w_ref, b_ref, out_ref):
  z = (jnp.dot(ea_ref[...], w_ref[...], preferred_element_type=jnp.float32)
       + b_ref[...])
  out_ref[...] = z.reshape(_G * _BER, D)


def _tc_edge_embed(ea8, w8, b8):
  return pl.pallas_call(
      _embed_body,
      grid=(E // _G // _BER,),
      in_specs=[
          pl.BlockSpec((_BER, _G * DE), lambda i: (i, 0)),
          pl.BlockSpec((_G * DE, _G * D), lambda i: (0, 0)),
          pl.BlockSpec((1, _G * D), lambda i: (0, 0)),
      ],
      out_specs=pl.BlockSpec((_G * _BER, D), lambda i: (i, 0)),
      out_shape=jax.ShapeDtypeStruct((E, D), jnp.float32),
  )(ea8, w8, b8.reshape(1, _G * D))


def _mlp_bn_body(relu_out, h_ref, a0_ref, a1_ref, sc_ref, w1_ref, b1_ref,
                 w2_ref, b2_ref, g_ref, bt_ref, out_ref):
  zin = sc_ref[...] * h_ref[...] + a0_ref[...] + a1_ref[...]
  t = jnp.maximum(
      jnp.dot(zin, w1_ref[...], preferred_element_type=jnp.float32)
      + b1_ref[...], 0.0)
  z = (jnp.dot(t, w2_ref[...], preferred_element_type=jnp.float32)
       + b2_ref[...])
  mean = jnp.mean(z, axis=0, keepdims=True)
  var = jnp.mean((z - mean) ** 2, axis=0, keepdims=True)
  zn = (z - mean) * lax.rsqrt(var + 1e-5) * g_ref[...] + bt_ref[...]
  if relu_out:
    zn = jnp.maximum(zn, 0.0)
  out_ref[...] = zn


def _tc_mlp_bn(h, agg2, scale, w1, b1, w2, b2, gamma, beta, relu_out):
  full = lambda shape: pl.BlockSpec(shape, lambda g: tuple(0 for _ in shape))
  return pl.pallas_call(
      functools.partial(_mlp_bn_body, relu_out),
      grid=(1,),
      in_specs=[
          full((N, D)),
          pl.BlockSpec((N, D), lambda g: (0, 0)),
          pl.BlockSpec((N, D), lambda g: (1, 0)),
          full((1, D)), full((D, H)), full((1, H)), full((H, D)),
          full((1, D)), full((1, D)), full((1, D)),
      ],
      out_specs=full((N, D)),
      out_shape=jax.ShapeDtypeStruct((N, D), jnp.float32),
  )(h, agg2, agg2,
    jnp.broadcast_to(scale.reshape(1, 1), (1, D)),
    w1, b1.reshape(1, H), w2, b2.reshape(1, D),
    gamma.reshape(1, D), beta.reshape(1, D))


def kernel(x, edge_index, edge_attr, W_e, b_e, eps, W1, b1, W2, b2,
           gamma, beta):
  zeros = jnp.zeros((ZROWS, D), dtype=jnp.float32)
  src = edge_index[0]
  dst = edge_index[1]

  ea8 = edge_attr.reshape(E // _G, _G * DE)
  eye8 = jnp.eye(_G, dtype=jnp.float32)
  h = x
  for l in range(L):
    w8_l = jnp.einsum("jm,kd->jkmd", eye8, W_e[l]).reshape(_G * DE, _G * D)
    b8_l = jnp.tile(b_e[l], _G)
    e = _tc_edge_embed(ea8, w8_l, b8_l)
    agg2 = _sc_message_aggregate(h, e, src, dst, zeros)
    h = _tc_mlp_bn(h, agg2, 1.0 + eps[l],
                   W1[l], b1[l], W2[l], b2[l], gamma[l], beta[l],
                   relu_out=(l < L - 1))
  return h


# speedup vs baseline: 1.0670x; 1.0004x over previous
import functools

import jax
import jax.numpy as jnp
from jax import lax
from jax.experimental import pallas as pl
from jax.experimental.pallas import tpu as pltpu
from jax.experimental.pallas import tpu_sc as plsc

N = 10000
E = 320000
D = 128
DE = 16
H = 2 * D
L = 2

NC = 2
NS = 16
NW = NC * NS
EW = E // NW
K = 40
NCHUNK = EW // K
SLOTS = 4
ISLOTS = 8
PEEL_LO, PEEL_HI = -4, 8
TAIL_LO, TAIL_HI = 240, 252
NGROUPS = (TAIL_LO - PEEL_HI) // ISLOTS
ZROWS = 640
AGG_ROWS = NS * ZROWS


def _sc_message_aggregate(h, e, src, dst, zeros):
  mesh = plsc.VectorSubcoreMesh(core_axis_name="c", subcore_axis_name="s")

  @functools.partial(
      pl.kernel,
      mesh=mesh,
      out_type=jax.ShapeDtypeStruct((2 * N, D), jnp.float32),
      scratch_types=[
          pltpu.VMEM((ISLOTS, 2, K), jnp.int32),
          pltpu.VMEM((SLOTS, K, D), jnp.float32),
          pltpu.VMEM((SLOTS, K, D), jnp.float32),
          pltpu.VMEM_SHARED((AGG_ROWS, D), jnp.float32),
          pltpu.SemaphoreType.DMA((ISLOTS,)),
          pltpu.SemaphoreType.DMA((SLOTS,)),
          pltpu.SemaphoreType.DMA((SLOTS,)),
          pltpu.SemaphoreType.DMA((SLOTS,)),
      ],
  )
  def k(h_hbm, e_hbm, src_hbm, dst_hbm, zero_hbm, out_hbm,
        idx_v, e_v, h_v, agg, sem_i, sem_e, sem_g, sem_s):
    c = lax.axis_index("c")
    s = lax.axis_index("s")
    wid = c * NS + s

    pltpu.sync_copy(zero_hbm, agg.at[pl.ds(s * ZROWS, ZROWS)])
    plsc.subcore_barrier()

    base = wid * EW

    def idx_start(j, m):
      off = base + j * K
      pltpu.async_copy(src_hbm.at[pl.ds(off, K)], idx_v.at[m % ISLOTS, 0],
                       sem_i.at[m % ISLOTS])
      pltpu.async_copy(dst_hbm.at[pl.ds(off, K)], idx_v.at[m % ISLOTS, 1],
                       sem_i.at[m % ISLOTS])

    def idx_wait(m):
      pltpu.make_async_copy(src_hbm.at[pl.ds(0, K)],
                            idx_v.at[m % ISLOTS, 0],
                            sem_i.at[m % ISLOTS]).wait()
      pltpu.make_async_copy(dst_hbm.at[pl.ds(0, K)],
                            idx_v.at[m % ISLOTS, 1],
                            sem_i.at[m % ISLOTS]).wait()

    def e_start(j, m):
      pltpu.async_copy(e_hbm.at[pl.ds(base + j * K, K)], e_v.at[m % SLOTS],
                       sem_e.at[m % SLOTS])

    def e_wait(m):
      pltpu.make_async_copy(e_hbm.at[pl.ds(0, K)], e_v.at[m % SLOTS],
                            sem_e.at[m % SLOTS]).wait()

    def gather_start(m):
      pltpu.async_copy(h_hbm.at[idx_v.at[m % ISLOTS, 0]], h_v.at[m % SLOTS],
                       sem_g.at[m % SLOTS])

    def gather_wait(m):
      pltpu.make_async_copy(h_hbm.at[idx_v.at[m % ISLOTS, 0]],
                            h_v.at[m % SLOTS], sem_g.at[m % SLOTS]).wait()

    def scatter_start(m):
      pltpu.async_copy(h_v.at[m % SLOTS], agg.at[idx_v.at[m % ISLOTS, 1]],
                       sem_s.at[m % SLOTS], add=True)

    def scatter_wait(m):
      pltpu.make_async_copy(h_v.at[m % SLOTS],
                            agg.at[idx_v.at[m % ISLOTS, 1]],
                            sem_s.at[m % SLOTS]).wait()

    def compute(m, unroll=4):
      b = m % SLOTS

      @plsc.parallel_loop(0, K, 1, unroll=unroll)
      def _(i):
        for r in range(D // 16):
          sl = pl.ds(r * 16, 16)
          h_v[b, i, sl] = jnp.maximum(h_v[b, i, sl] + e_v[b, i, sl], 0.0)

    def step(j, m, guard):
      def ok(chunk):
        return guard is None or 0 <= chunk < NCHUNK

      if ok(j - 2):
        scatter_wait(m - 2)
      if ok(j + 4):
        idx_start(j + 4, m + 4)
      if ok(j + 2):
        e_start(j + 2, m + 2)
        idx_wait(m + 2)
        gather_start(m + 2)
      if ok(j):
        gather_wait(m)
        e_wait(m)
        compute(m, unroll=4 if guard is None else 1)
        scatter_start(m)

    for t in range(PEEL_LO, PEEL_HI):
      step(t, t % ISLOTS, guard=t)

    def group(g, carry):
      j0 = PEEL_HI + ISLOTS * g
      for i in range(ISLOTS):
        step(j0 + i, (PEEL_HI + i) % ISLOTS, guard=None)
      return carry

    lax.fori_loop(0, NGROUPS, group, 0)

    for t in range(TAIL_LO, TAIL_HI):
      step(t, t % ISLOTS, guard=t)

    plsc.subcore_barrier()

    @pl.when(s < NS - 1)
    def _():
      pltpu.sync_copy(agg.at[pl.ds(s * ZROWS, ZROWS)],
                      out_hbm.at[pl.ds(c * N + s * ZROWS, ZROWS)])

    @pl.when(s == NS - 1)
    def _():
      last = N - (NS - 1) * ZROWS
      pltpu.sync_copy(agg.at[pl.ds((NS - 1) * ZROWS, last)],
                      out_hbm.at[pl.ds(c * N + (NS - 1) * ZROWS, last)])

  return k(h, e, src, dst, zeros)


_G = 128 // DE
_BER = 2000


def _embed_body(ea_ref, w_ref, b_ref, out_ref):
  z = (jnp.dot(ea_ref[...], w_ref[...], preferred_element_type=jnp.float32)
       + b_ref[...])
  out_ref[...] = z.reshape(_G * _BER, D)


def _tc_edge_embed(ea8, w8, b8):
  return pl.pallas_call(
      _embed_body,
      grid=(E // _G // _BER,),
      in_specs=[
          pl.BlockSpec((_BER, _G * DE), lambda i: (i, 0)),
          pl.BlockSpec((_G * DE, _G * D), lambda i: (0, 0)),
          pl.BlockSpec((1, _G * D), lambda i: (0, 0)),
      ],
      out_specs=pl.BlockSpec((_G * _BER, D), lambda i: (i, 0)),
      out_shape=jax.ShapeDtypeStruct((E, D), jnp.float32),
  )(ea8, w8, b8.reshape(1, _G * D))


def _mlp_bn_body(relu_out, h_ref, a0_ref, a1_ref, sc_ref, w1_ref, b1_ref,
                 w2_ref, b2_ref, g_ref, bt_ref, out_ref):
  zin = sc_ref[...] * h_ref[...] + a0_ref[...] + a1_ref[...]
  t = jnp.maximum(
      jnp.dot(zin, w1_ref[...], preferred_element_type=jnp.float32)
      + b1_ref[...], 0.0)
  z = (jnp.dot(t, w2_ref[...], preferred_element_type=jnp.float32)
       + b2_ref[...])
  mean = jnp.mean(z, axis=0, keepdims=True)
  var = jnp.mean((z - mean) ** 2, axis=0, keepdims=True)
  zn = (z - mean) * lax.rsqrt(var + 1e-5) * g_ref[...] + bt_ref[...]
  if relu_out:
    zn = jnp.maximum(zn, 0.0)
  out_ref[...] = zn


def _tc_mlp_bn(h, agg2, scale, w1, b1, w2, b2, gamma, beta, relu_out):
  full = lambda shape: pl.BlockSpec(shape, lambda g: tuple(0 for _ in shape))
  return pl.pallas_call(
      functools.partial(_mlp_bn_body, relu_out),
      grid=(1,),
      in_specs=[
          full((N, D)),
          pl.BlockSpec((N, D), lambda g: (0, 0)),
          pl.BlockSpec((N, D), lambda g: (1, 0)),
          full((1, D)), full((D, H)), full((1, H)), full((H, D)),
          full((1, D)), full((1, D)), full((1, D)),
      ],
      out_specs=full((N, D)),
      out_shape=jax.ShapeDtypeStruct((N, D), jnp.float32),
  )(h, agg2, agg2,
    jnp.broadcast_to(scale.reshape(1, 1), (1, D)),
    w1, b1.reshape(1, H), w2, b2.reshape(1, D),
    gamma.reshape(1, D), beta.reshape(1, D))


def kernel(x, edge_index, edge_attr, W_e, b_e, eps, W1, b1, W2, b2,
           gamma, beta):
  zeros = jnp.zeros((ZROWS, D), dtype=jnp.float32)
  src = edge_index[0]
  dst = edge_index[1]

  ea8 = edge_attr.reshape(E // _G, _G * DE)
  eye8 = jnp.eye(_G, dtype=jnp.float32)
  h = x
  for l in range(L):
    w8_l = jnp.einsum("jm,kd->jkmd", eye8, W_e[l]).reshape(_G * DE, _G * D)
    b8_l = jnp.tile(b_e[l], _G)
    e = _tc_edge_embed(ea8, w8_l, b8_l)
    agg2 = _sc_message_aggregate(h, e, src, dst, zeros)
    h = _tc_mlp_bn(h, agg2, 1.0 + eps[l],
                   W1[l], b1[l], W2[l], b2[l], gamma[l], beta[l],
                   relu_out=(l < L - 1))
  return h
